# Initial kernel scaffold; baseline (speedup 1.0000x reference)
import jax
import jax.numpy as jnp
from jax.experimental import pallas as pl


def kernel(edge_index, edge_type, weight1, root1, bias1, weight2, root2, bias2):
    raise NotImplementedError("write your pallas kernel here")


# speedup vs baseline: 4.4520x; 4.4520x over previous
import functools

import jax
import jax.numpy as jnp
from jax import lax
from jax.experimental import pallas as pl
from jax.experimental.pallas import tpu as pltpu
from jax.experimental.pallas import tpu_sc as plsc

_NC = 2
_NS = 16
_NW = _NC * _NS
_LN = 16
_B = 80


def _mesh():
    return plsc.VectorSubcoreMesh(
        core_axis_name="c", subcore_axis_name="s", num_cores=_NC,
        num_subcores=_NS)


def _wid(cid, sid):
    return sid * _NC + cid


def _make_hist(N, R, E):
    EP = E // _NW
    NR = N * R
    ZS = NR // _NS

    def body(src_hbm, dst_hbm, typ_hbm, gidx_hbm, seg_hbm, cnt_hbm,
             src_v, dst_v, typ_v, gidx_v, seg_v, idx_v, ones_v, z_v, cnt_sh):
        cid = lax.axis_index("c")
        sid = lax.axis_index("s")
        base = _wid(cid, sid) * EP

        pltpu.sync_copy(src_hbm.at[pl.ds(base, EP)], src_v)
        pltpu.sync_copy(dst_hbm.at[pl.ds(base, EP)], dst_v)
        pltpu.sync_copy(typ_hbm.at[pl.ds(base, EP)], typ_v)

        def compute(k, carry):
            s = src_v[pl.ds(k * _LN, _LN)]
            d = dst_v[pl.ds(k * _LN, _LN)]
            t = typ_v[pl.ds(k * _LN, _LN)]
            seg_v[pl.ds(k * _LN, _LN)] = d * R + t
            gidx_v[pl.ds(k * _LN, _LN)] = t * N + s
            return carry
        lax.fori_loop(0, EP // _LN, compute, 0)

        pltpu.sync_copy(gidx_v, gidx_hbm.at[pl.ds(base, EP)])
        pltpu.sync_copy(seg_v, seg_hbm.at[pl.ds(base, EP)])

        def zfill(k, carry):
            z_v[pl.ds(k * _LN, _LN)] = jnp.zeros((_LN,), jnp.float32)
            return carry
        lax.fori_loop(0, ZS // _LN, zfill, 0)
        for u in range(_B // _LN):
            ones_v[pl.ds(u * _LN, _LN)] = jnp.ones((_LN,), jnp.float32)

        pltpu.sync_copy(z_v, cnt_sh.at[pl.ds(sid * ZS, ZS)])
        plsc.subcore_barrier()

        def scat(j, carry):
            for u in range(_B // _LN):
                idx_v[pl.ds(u * _LN, _LN)] = seg_v[pl.ds(j * _B + u * _LN,
                                                         _LN)]
            pltpu.sync_copy(ones_v, cnt_sh.at[idx_v], add=True)
            return carry
        lax.fori_loop(0, EP // _B, scat, 0)
        plsc.subcore_barrier()

        pltpu.sync_copy(cnt_sh.at[pl.ds(sid * ZS, ZS)], z_v)
        pltpu.sync_copy(z_v, cnt_hbm.at[pl.ds(cid * NR + sid * ZS, ZS)])

    return pl.kernel(
        body,
        out_type=(
            jax.ShapeDtypeStruct((E,), jnp.int32),
            jax.ShapeDtypeStruct((E,), jnp.int32),
            jax.ShapeDtypeStruct((_NC * NR,), jnp.float32),
        ),
        mesh=_mesh(),
        scratch_types=[
            pltpu.VMEM((EP,), jnp.int32),
            pltpu.VMEM((EP,), jnp.int32),
            pltpu.VMEM((EP,), jnp.int32),
            pltpu.VMEM((EP,), jnp.int32),
            pltpu.VMEM((EP,), jnp.int32),
            pltpu.VMEM((_B,), jnp.int32),
            pltpu.VMEM((_B,), jnp.float32),
            pltpu.VMEM((ZS,), jnp.float32),
            pltpu.VMEM_SHARED((NR,), jnp.float32),
        ],
    )


def _make_w(N, R, E):
    EP = E // _NW

    def body(seg_hbm, inv_hbm, w_hbm, seg_v, w_v, idx_v):
        cid = lax.axis_index("c")
        sid = lax.axis_index("s")
        base = _wid(cid, sid) * EP
        pltpu.sync_copy(seg_hbm.at[pl.ds(base, EP)], seg_v)

        def gath(j, carry):
            for u in range(_B // _LN):
                idx_v[pl.ds(u * _LN, _LN)] = seg_v[pl.ds(j * _B + u * _LN,
                                                         _LN)]
            pltpu.sync_copy(inv_hbm.at[idx_v], w_v.at[pl.ds(j * _B, _B)])
            return carry
        lax.fori_loop(0, EP // _B, gath, 0)
        pltpu.sync_copy(w_v, w_hbm.at[pl.ds(base, EP)])

    return pl.kernel(
        body,
        out_type=jax.ShapeDtypeStruct((E,), jnp.float32),
        mesh=_mesh(),
        scratch_types=[
            pltpu.VMEM((EP,), jnp.int32),
            pltpu.VMEM((EP,), jnp.float32),
            pltpu.VMEM((_B,), jnp.int32),
        ],
    )


def _make_layer(N, R, H, E):
    EP = E // _NW
    RP = N // _NS
    ZB = 25
    DB = 40

    def body(tab_hbm, gidx_hbm, dst_hbm, w_hbm, part_hbm,
             gidx_v, dst_v, w_v, gi_v, di_v, rows_v, z2_v, d_v, acc_sh):
        cid = lax.axis_index("c")
        sid = lax.axis_index("s")
        base = _wid(cid, sid) * EP

        pltpu.sync_copy(gidx_hbm.at[pl.ds(base, EP)], gidx_v)
        pltpu.sync_copy(dst_hbm.at[pl.ds(base, EP)], dst_v)
        pltpu.sync_copy(w_hbm.at[pl.ds(base, EP)], w_v)

        for r in range(ZB):
            for k in range(H // _LN):
                z2_v[r, pl.ds(k * _LN, _LN)] = jnp.zeros((_LN,), jnp.float32)

        def zero(m, carry):
            pltpu.sync_copy(z2_v, acc_sh.at[pl.ds(sid * RP + m * ZB, ZB), :])
            return carry
        lax.fori_loop(0, RP // ZB, zero, 0)
        plsc.subcore_barrier()

        def step(j, carry):
            for u in range(_B // _LN):
                gi_v[pl.ds(u * _LN, _LN)] = gidx_v[pl.ds(j * _B + u * _LN,
                                                         _LN)]
                di_v[pl.ds(u * _LN, _LN)] = dst_v[pl.ds(j * _B + u * _LN,
                                                        _LN)]
            pltpu.sync_copy(tab_hbm.at[gi_v], rows_v)

            for u in range(_B // _LN):
                w16 = w_v[pl.ds(j * _B + u * _LN, _LN)]
                for i in range(_LN):
                    ws = w16[i]
                    row = u * _LN + i
                    for k in range(H // _LN):
                        rows_v[row, pl.ds(k * _LN, _LN)] = (
                            rows_v[row, pl.ds(k * _LN, _LN)] * ws)
            pltpu.sync_copy(rows_v, acc_sh.at[di_v], add=True)
            return carry
        lax.fori_loop(0, EP // _B, step, 0)
        plsc.subcore_barrier()

        nchunk = N // DB
        per_tile = (nchunk + _NS - 1) // _NS

        def dump(m, carry):
            cidx = sid * per_tile + m

            @pl.when(cidx < nchunk)
            def _():
                row0 = pl.multiple_of(cidx * DB, 8)
                pltpu.sync_copy(acc_sh.at[pl.ds(row0, DB), :], d_v)
                pltpu.sync_copy(d_v, part_hbm.at[pl.ds(cid * N + row0,
                                                       DB), :])
            return carry
        lax.fori_loop(0, per_tile, dump, 0)

    return pl.kernel(
        body,
        out_type=jax.ShapeDtypeStruct((_NC * N, H), jnp.float32),
        mesh=_mesh(),
        scratch_types=[
            pltpu.VMEM((EP,), jnp.int32),
            pltpu.VMEM((EP,), jnp.int32),
            pltpu.VMEM((EP,), jnp.float32),
            pltpu.VMEM((_B,), jnp.int32),
            pltpu.VMEM((_B,), jnp.int32),
            pltpu.VMEM((_B, H), jnp.float32),
            pltpu.VMEM((ZB, H), jnp.float32),
            pltpu.VMEM((DB, H), jnp.float32),
            pltpu.VMEM_SHARED((N, H), jnp.float32),
        ],
    )


def _inv_body(cnt_ref, inv_ref):
    c = cnt_ref[0] + cnt_ref[1]
    inv_ref[...] = 1.0 / jnp.maximum(c, 1.0)


def _make_act(N, R, H, BN):
    def body(p0, p1, r1, b1, w2, x_ref, t_ref):
        v = p0[...] + p1[...] + r1[...] + b1[...]
        x = jnp.where(v > 0, v, 0.01 * v)
        x_ref[...] = x
        for r in range(R):
            t_ref[r] = jnp.dot(x, w2[r], preferred_element_type=jnp.float32)

    return pl.pallas_call(
        body,
        grid=(N // BN,),
        in_specs=[
            pl.BlockSpec((BN, H), lambda i: (i, 0)),
            pl.BlockSpec((BN, H), lambda i: (i, 0)),
            pl.BlockSpec((BN, H), lambda i: (i, 0)),
            pl.BlockSpec((1, H), lambda i: (0, 0)),
            pl.BlockSpec((R, H, H), lambda i: (0, 0, 0)),
        ],
        out_specs=[
            pl.BlockSpec((BN, H), lambda i: (i, 0)),
            pl.BlockSpec((R, BN, H), lambda i: (0, i, 0)),
        ],
        out_shape=[
            jax.ShapeDtypeStruct((N, H), jnp.float32),
            jax.ShapeDtypeStruct((R, N, H), jnp.float32),
        ],
    )


def _make_out(N, H, BN):
    def body(p0, p1, x, r2, b2, o_ref):
        o_ref[...] = (p0[...] + p1[...] + b2[...]
                      + jnp.dot(x[...], r2[...],
                                preferred_element_type=jnp.float32))

    return pl.pallas_call(
        body,
        grid=(N // BN,),
        in_specs=[
            pl.BlockSpec((BN, H), lambda i: (i, 0)),
            pl.BlockSpec((BN, H), lambda i: (i, 0)),
            pl.BlockSpec((BN, H), lambda i: (i, 0)),
            pl.BlockSpec((H, H), lambda i: (0, 0)),
            pl.BlockSpec((1, H), lambda i: (0, 0)),
        ],
        out_specs=pl.BlockSpec((BN, H), lambda i: (i, 0)),
        out_shape=jax.ShapeDtypeStruct((N, H), jnp.float32),
    )


def kernel(edge_index, edge_type, weight1, root1, bias1, weight2, root2,
           bias2):
    R, N, H = weight1.shape
    E = edge_type.shape[0]
    NR = N * R
    assert E % (_NW * _B) == 0 and N % (_NS * 25) == 0 and NR % _NS == 0

    src = edge_index[0].astype(jnp.int32)
    dst = edge_index[1].astype(jnp.int32)
    typ = edge_type.astype(jnp.int32)

    k_hist = _make_hist(N, R, E)
    gidx, seg, cnt = k_hist(src, dst, typ)

    k_inv = pl.pallas_call(
        _inv_body,
        out_shape=jax.ShapeDtypeStruct((NR // 128, 128), jnp.float32))
    inv = k_inv(cnt.reshape(_NC, NR // 128, 128)).reshape(NR)

    k_w = _make_w(N, R, E)
    w = k_w(seg, inv)

    k_layer = _make_layer(N, R, H, E)
    part1 = k_layer(weight1.reshape(NR, H), gidx, dst, w)

    k_act = _make_act(N, R, H, 400)
    x, t = k_act(part1[:N], part1[N:], root1, bias1.reshape(1, H), weight2)

    part2 = k_layer(t.reshape(NR, H), gidx, dst, w)

    k_out = _make_out(N, H, 1000)
    return k_out(part2[:N], part2[N:], x, root2, bias2.reshape(1, H))


# speedup vs baseline: 4.4538x; 1.0004x over previous
import functools

import jax
import jax.numpy as jnp
from jax import lax
from jax.experimental import pallas as pl
from jax.experimental.pallas import tpu as pltpu
from jax.experimental.pallas import tpu_sc as plsc

_NC = 2
_NS = 16
_NW = _NC * _NS
_LN = 16
_B = 80


def _mesh():
    return plsc.VectorSubcoreMesh(
        core_axis_name="c", subcore_axis_name="s", num_cores=_NC,
        num_subcores=_NS)


def _wid(cid, sid):
    return sid * _NC + cid


def _make_hist(N, R, E):
    EP = E // _NW
    NR = N * R
    ZS = NR // _NS

    def body(src_hbm, dst_hbm, typ_hbm, gidx_hbm, seg_hbm, cnt_hbm,
             src_v, dst_v, typ_v, gidx_v, seg_v, idx_v, ones_v, z_v, cnt_sh):
        cid = lax.axis_index("c")
        sid = lax.axis_index("s")
        base = _wid(cid, sid) * EP

        pltpu.sync_copy(src_hbm.at[pl.ds(base, EP)], src_v)
        pltpu.sync_copy(dst_hbm.at[pl.ds(base, EP)], dst_v)
        pltpu.sync_copy(typ_hbm.at[pl.ds(base, EP)], typ_v)

        def compute(k, carry):
            s = src_v[pl.ds(k * _LN, _LN)]
            d = dst_v[pl.ds(k * _LN, _LN)]
            t = typ_v[pl.ds(k * _LN, _LN)]
            seg_v[pl.ds(k * _LN, _LN)] = d * R + t
            gidx_v[pl.ds(k * _LN, _LN)] = t * N + s
            return carry
        lax.fori_loop(0, EP // _LN, compute, 0)

        pltpu.sync_copy(gidx_v, gidx_hbm.at[pl.ds(base, EP)])
        pltpu.sync_copy(seg_v, seg_hbm.at[pl.ds(base, EP)])

        def zfill(k, carry):
            z_v[pl.ds(k * _LN, _LN)] = jnp.zeros((_LN,), jnp.float32)
            return carry
        lax.fori_loop(0, ZS // _LN, zfill, 0)
        for u in range(_B // _LN):
            ones_v[pl.ds(u * _LN, _LN)] = jnp.ones((_LN,), jnp.float32)

        pltpu.sync_copy(z_v, cnt_sh.at[pl.ds(sid * ZS, ZS)])
        plsc.subcore_barrier()

        def scat(j, carry):
            for u in range(_B // _LN):
                idx_v[pl.ds(u * _LN, _LN)] = seg_v[pl.ds(j * _B + u * _LN,
                                                         _LN)]
            pltpu.sync_copy(ones_v, cnt_sh.at[idx_v], add=True)
            return carry
        lax.fori_loop(0, EP // _B, scat, 0)
        plsc.subcore_barrier()

        pltpu.sync_copy(cnt_sh.at[pl.ds(sid * ZS, ZS)], z_v)
        pltpu.sync_copy(z_v, cnt_hbm.at[pl.ds(cid * NR + sid * ZS, ZS)])

    return pl.kernel(
        body,
        out_type=(
            jax.ShapeDtypeStruct((E,), jnp.int32),
            jax.ShapeDtypeStruct((E,), jnp.int32),
            jax.ShapeDtypeStruct((_NC * NR,), jnp.float32),
        ),
        mesh=_mesh(),
        scratch_types=[
            pltpu.VMEM((EP,), jnp.int32),
            pltpu.VMEM((EP,), jnp.int32),
            pltpu.VMEM((EP,), jnp.int32),
            pltpu.VMEM((EP,), jnp.int32),
            pltpu.VMEM((EP,), jnp.int32),
            pltpu.VMEM((_B,), jnp.int32),
            pltpu.VMEM((_B,), jnp.float32),
            pltpu.VMEM((ZS,), jnp.float32),
            pltpu.VMEM_SHARED((NR,), jnp.float32),
        ],
    )


def _make_w(N, R, E):
    EP = E // _NW

    def body(seg_hbm, inv_hbm, w_hbm, seg_v, w_v, idx_v):
        cid = lax.axis_index("c")
        sid = lax.axis_index("s")
        base = _wid(cid, sid) * EP
        pltpu.sync_copy(seg_hbm.at[pl.ds(base, EP)], seg_v)

        def gath(j, carry):
            for u in range(_B // _LN):
                idx_v[pl.ds(u * _LN, _LN)] = seg_v[pl.ds(j * _B + u * _LN,
                                                         _LN)]
            pltpu.sync_copy(inv_hbm.at[idx_v], w_v.at[pl.ds(j * _B, _B)])
            return carry
        lax.fori_loop(0, EP // _B, gath, 0)
        pltpu.sync_copy(w_v, w_hbm.at[pl.ds(base, EP)])

    return pl.kernel(
        body,
        out_type=jax.ShapeDtypeStruct((E,), jnp.float32),
        mesh=_mesh(),
        scratch_types=[
            pltpu.VMEM((EP,), jnp.int32),
            pltpu.VMEM((EP,), jnp.float32),
            pltpu.VMEM((_B,), jnp.int32),
        ],
    )


def _make_layer(N, R, H, E):
    EP = E // _NW
    RP = N // _NS
    ZB = 25
    DB = 40

    def body(tab_hbm, gidx_hbm, dst_hbm, w_hbm, part_hbm,
             gidx_v, dst_v, w_v, rows_v, gi_v, di_v, z2_v, d_v, acc_sh):
        cid = lax.axis_index("c")
        sid = lax.axis_index("s")
        base = _wid(cid, sid) * EP

        pltpu.sync_copy(gidx_hbm.at[pl.ds(base, EP)], gidx_v)
        pltpu.sync_copy(dst_hbm.at[pl.ds(base, EP)], dst_v)
        pltpu.sync_copy(w_hbm.at[pl.ds(base, EP)], w_v)

        for r in range(ZB):
            for k in range(H // _LN):
                z2_v[r, pl.ds(k * _LN, _LN)] = jnp.zeros((_LN,), jnp.float32)

        def zero(m, carry):
            pltpu.sync_copy(z2_v, acc_sh.at[pl.ds(sid * RP + m * ZB, ZB), :])
            return carry
        lax.fori_loop(0, RP // ZB, zero, 0)
        plsc.subcore_barrier()

        def step(j, carry):
            for u in range(_B // _LN):
                gi_v[pl.ds(u * _LN, _LN)] = gidx_v[pl.ds(j * _B + u * _LN,
                                                         _LN)]
                di_v[pl.ds(u * _LN, _LN)] = dst_v[pl.ds(j * _B + u * _LN,
                                                        _LN)]
            pltpu.sync_copy(tab_hbm.at[gi_v], rows_v)

            for u in range(_B // _LN):
                w16 = w_v[pl.ds(j * _B + u * _LN, _LN)]
                for i in range(_LN):
                    ws = w16[i]
                    row = u * _LN + i
                    for k in range(H // _LN):
                        rows_v[row, pl.ds(k * _LN, _LN)] = (
                            rows_v[row, pl.ds(k * _LN, _LN)] * ws)
            pltpu.sync_copy(rows_v, acc_sh.at[di_v], add=True)
            return carry
        lax.fori_loop(0, EP // _B, step, 0)
        plsc.subcore_barrier()

        nchunk = N // DB
        per_tile = (nchunk + _NS - 1) // _NS

        def dump(m, carry):
            cidx = sid * per_tile + m

            @pl.when(cidx < nchunk)
            def _():
                row0 = pl.multiple_of(cidx * DB, 8)
                pltpu.sync_copy(acc_sh.at[pl.ds(row0, DB), :], d_v)
                pltpu.sync_copy(d_v, part_hbm.at[pl.ds(cid * N + row0,
                                                       DB), :])
            return carry
        lax.fori_loop(0, per_tile, dump, 0)

    return pl.kernel(
        body,
        out_type=jax.ShapeDtypeStruct((_NC * N, H), jnp.float32),
        mesh=_mesh(),
        scratch_types=[
            pltpu.VMEM((EP,), jnp.int32),
            pltpu.VMEM((EP,), jnp.int32),
            pltpu.VMEM((EP,), jnp.float32),
            pltpu.VMEM((_B, H), jnp.float32),
            pltpu.VMEM((_B,), jnp.int32),
            pltpu.VMEM((_B,), jnp.int32),
            pltpu.VMEM((ZB, H), jnp.float32),
            pltpu.VMEM((DB, H), jnp.float32),
            pltpu.VMEM_SHARED((N, H), jnp.float32),
        ],
    )


def _inv_body(cnt_ref, inv_ref):
    c = cnt_ref[0] + cnt_ref[1]
    inv_ref[...] = 1.0 / jnp.maximum(c, 1.0)


def _make_act(N, R, H, BN):
    def body(p0, p1, r1, b1, w2, x_ref, t_ref):
        v = p0[...] + p1[...] + r1[...] + b1[...]
        x = jnp.where(v > 0, v, 0.01 * v)
        x_ref[...] = x
        for r in range(R):
            t_ref[r] = jnp.dot(x, w2[r], preferred_element_type=jnp.float32)

    return pl.pallas_call(
        body,
        grid=(N // BN,),
        in_specs=[
            pl.BlockSpec((BN, H), lambda i: (i, 0)),
            pl.BlockSpec((BN, H), lambda i: (i, 0)),
            pl.BlockSpec((BN, H), lambda i: (i, 0)),
            pl.BlockSpec((1, H), lambda i: (0, 0)),
            pl.BlockSpec((R, H, H), lambda i: (0, 0, 0)),
        ],
        out_specs=[
            pl.BlockSpec((BN, H), lambda i: (i, 0)),
            pl.BlockSpec((R, BN, H), lambda i: (0, i, 0)),
        ],
        out_shape=[
            jax.ShapeDtypeStruct((N, H), jnp.float32),
            jax.ShapeDtypeStruct((R, N, H), jnp.float32),
        ],
    )


def _make_out(N, H, BN):
    def body(p0, p1, x, r2, b2, o_ref):
        o_ref[...] = (p0[...] + p1[...] + b2[...]
                      + jnp.dot(x[...], r2[...],
                                preferred_element_type=jnp.float32))

    return pl.pallas_call(
        body,
        grid=(N // BN,),
        in_specs=[
            pl.BlockSpec((BN, H), lambda i: (i, 0)),
            pl.BlockSpec((BN, H), lambda i: (i, 0)),
            pl.BlockSpec((BN, H), lambda i: (i, 0)),
            pl.BlockSpec((H, H), lambda i: (0, 0)),
            pl.BlockSpec((1, H), lambda i: (0, 0)),
        ],
        out_specs=pl.BlockSpec((BN, H), lambda i: (i, 0)),
        out_shape=jax.ShapeDtypeStruct((N, H), jnp.float32),
    )


def kernel(edge_index, edge_type, weight1, root1, bias1, weight2, root2,
           bias2):
    R, N, H = weight1.shape
    E = edge_type.shape[0]
    NR = N * R
    assert E % (_NW * _B) == 0 and N % (_NS * 25) == 0 and NR % _NS == 0

    src = edge_index[0].astype(jnp.int32)
    dst = edge_index[1].astype(jnp.int32)
    typ = edge_type.astype(jnp.int32)

    k_hist = _make_hist(N, R, E)
    gidx, seg, cnt = k_hist(src, dst, typ)

    k_inv = pl.pallas_call(
        _inv_body,
        out_shape=jax.ShapeDtypeStruct((NR // 128, 128), jnp.float32))
    inv = k_inv(cnt.reshape(_NC, NR // 128, 128)).reshape(NR)

    k_w = _make_w(N, R, E)
    w = k_w(seg, inv)

    k_layer = _make_layer(N, R, H, E)
    part1 = k_layer(weight1.reshape(NR, H), gidx, dst, w)

    k_act = _make_act(N, R, H, 400)
    x, t = k_act(part1[:N], part1[N:], root1, bias1.reshape(1, H), weight2)

    part2 = k_layer(t.reshape(NR, H), gidx, dst, w)

    k_out = _make_out(N, H, 1000)
    return k_out(part2[:N], part2[N:], x, root2, bias2.reshape(1, H))


# speedup vs baseline: 4.4557x; 1.0004x over previous
import jax
import jax.numpy as jnp
from jax import lax
from jax.experimental import pallas as pl
from jax.experimental.pallas import tpu as pltpu
from jax.experimental.pallas import tpu_sc as plsc

_NC = 2
_NS = 16
_NW = _NC * _NS
_LN = 16
_B = 80


def _mesh():
    return plsc.VectorSubcoreMesh(
        core_axis_name="c", subcore_axis_name="s", num_cores=_NC,
        num_subcores=_NS)


def _wid(cid, sid):
    return sid * _NC + cid


def _make_hist(N, R, E):
    EP = E // _NW
    NR = N * R
    ZS = NR // _NS

    def body(src_hbm, dst_hbm, typ_hbm, gidx_hbm, seg_hbm, cnt_hbm,
             src_v, dst_v, typ_v, gidx_v, seg_v, idx_v, ones_v, z_v, cnt_sh):
        cid = lax.axis_index("c")
        sid = lax.axis_index("s")
        base = _wid(cid, sid) * EP

        pltpu.sync_copy(src_hbm.at[pl.ds(base, EP)], src_v)
        pltpu.sync_copy(dst_hbm.at[pl.ds(base, EP)], dst_v)
        pltpu.sync_copy(typ_hbm.at[pl.ds(base, EP)], typ_v)

        def compute(k, carry):
            s = src_v[pl.ds(k * _LN, _LN)]
            d = dst_v[pl.ds(k * _LN, _LN)]
            t = typ_v[pl.ds(k * _LN, _LN)]
            seg_v[pl.ds(k * _LN, _LN)] = d * R + t
            gidx_v[pl.ds(k * _LN, _LN)] = t * N + s
            return carry
        lax.fori_loop(0, EP // _LN, compute, 0)

        pltpu.sync_copy(gidx_v, gidx_hbm.at[pl.ds(base, EP)])
        pltpu.sync_copy(seg_v, seg_hbm.at[pl.ds(base, EP)])

        def zfill(k, carry):
            z_v[pl.ds(k * _LN, _LN)] = jnp.zeros((_LN,), jnp.float32)
            return carry
        lax.fori_loop(0, ZS // _LN, zfill, 0)
        for u in range(_B // _LN):
            ones_v[pl.ds(u * _LN, _LN)] = jnp.ones((_LN,), jnp.float32)

        pltpu.sync_copy(z_v, cnt_sh.at[pl.ds(sid * ZS, ZS)])
        plsc.subcore_barrier()

        def scat(j, carry):
            for u in range(_B // _LN):
                idx_v[pl.ds(u * _LN, _LN)] = seg_v[pl.ds(j * _B + u * _LN,
                                                         _LN)]
            pltpu.sync_copy(ones_v, cnt_sh.at[idx_v], add=True)
            return carry
        lax.fori_loop(0, EP // _B, scat, 0)
        plsc.subcore_barrier()

        pltpu.sync_copy(cnt_sh.at[pl.ds(sid * ZS, ZS)], z_v)
        pltpu.sync_copy(z_v, cnt_hbm.at[pl.ds(cid * NR + sid * ZS, ZS)])

    return pl.kernel(
        body,
        out_type=(
            jax.ShapeDtypeStruct((E,), jnp.int32),
            jax.ShapeDtypeStruct((E,), jnp.int32),
            jax.ShapeDtypeStruct((_NC * NR,), jnp.float32),
        ),
        mesh=_mesh(),
        scratch_types=[
            pltpu.VMEM((EP,), jnp.int32),
            pltpu.VMEM((EP,), jnp.int32),
            pltpu.VMEM((EP,), jnp.int32),
            pltpu.VMEM((EP,), jnp.int32),
            pltpu.VMEM((EP,), jnp.int32),
            pltpu.VMEM((_B,), jnp.int32),
            pltpu.VMEM((_B,), jnp.float32),
            pltpu.VMEM((ZS,), jnp.float32),
            pltpu.VMEM_SHARED((NR,), jnp.float32),
        ],
    )


def _make_w(N, R, E):
    EP = E // _NW

    def body(seg_hbm, inv_hbm, w_hbm, seg_v, w_v, idx_v):
        cid = lax.axis_index("c")
        sid = lax.axis_index("s")
        base = _wid(cid, sid) * EP
        pltpu.sync_copy(seg_hbm.at[pl.ds(base, EP)], seg_v)

        def gath(j, carry):
            for u in range(_B // _LN):
                idx_v[pl.ds(u * _LN, _LN)] = seg_v[pl.ds(j * _B + u * _LN,
                                                         _LN)]
            pltpu.sync_copy(inv_hbm.at[idx_v], w_v.at[pl.ds(j * _B, _B)])
            return carry
        lax.fori_loop(0, EP // _B, gath, 0)
        pltpu.sync_copy(w_v, w_hbm.at[pl.ds(base, EP)])

    return pl.kernel(
        body,
        out_type=jax.ShapeDtypeStruct((E,), jnp.float32),
        mesh=_mesh(),
        scratch_types=[
            pltpu.VMEM((EP,), jnp.int32),
            pltpu.VMEM((EP,), jnp.float32),
            pltpu.VMEM((_B,), jnp.int32),
        ],
    )


def _make_layer(N, R, H, E):
    EP = E // _NW
    RP = N // _NS
    ZB = 25
    DB = 40

    def body(tab_hbm, gidx_hbm, dst_hbm, w_hbm, part_hbm,
             gidx_v, dst_v, w_v, rows_v, gi_v, di_v, z2_v, d_v, acc_sh):
        cid = lax.axis_index("c")
        sid = lax.axis_index("s")
        base = _wid(cid, sid) * EP

        pltpu.sync_copy(gidx_hbm.at[pl.ds(base, EP)], gidx_v)
        pltpu.sync_copy(dst_hbm.at[pl.ds(base, EP)], dst_v)
        pltpu.sync_copy(w_hbm.at[pl.ds(base, EP)], w_v)

        for r in range(ZB):
            for k in range(H // _LN):
                z2_v[r, pl.ds(k * _LN, _LN)] = jnp.zeros((_LN,), jnp.float32)

        def zero(m, carry):
            pltpu.sync_copy(z2_v, acc_sh.at[pl.ds(sid * RP + m * ZB, ZB), :])
            return carry
        lax.fori_loop(0, RP // ZB, zero, 0)
        plsc.subcore_barrier()

        def step(j, carry):
            for u in range(_B // _LN):
                gi_v[pl.ds(u * _LN, _LN)] = gidx_v[pl.ds(j * _B + u * _LN,
                                                         _LN)]
                di_v[pl.ds(u * _LN, _LN)] = dst_v[pl.ds(j * _B + u * _LN,
                                                        _LN)]
            pltpu.sync_copy(tab_hbm.at[gi_v], rows_v)

            for u in range(_B // _LN):
                w16 = w_v[pl.ds(j * _B + u * _LN, _LN)]
                for i in range(_LN):
                    ws = w16[i]
                    row = u * _LN + i
                    for k in range(H // _LN):
                        rows_v[row, pl.ds(k * _LN, _LN)] = (
                            rows_v[row, pl.ds(k * _LN, _LN)] * ws)
            pltpu.sync_copy(rows_v, acc_sh.at[di_v], add=True)
            return carry
        lax.fori_loop(0, EP // _B, step, 0)
        plsc.subcore_barrier()

        nchunk = N // DB
        per_tile = (nchunk + _NS - 1) // _NS

        def dump(m, carry):
            cidx = sid * per_tile + m

            @pl.when(cidx < nchunk)
            def _():
                row0 = pl.multiple_of(cidx * DB, 8)
                pltpu.sync_copy(acc_sh.at[pl.ds(row0, DB), :], d_v)
                pltpu.sync_copy(d_v, part_hbm.at[pl.ds(cid * N + row0,
                                                       DB), :])
            return carry
        lax.fori_loop(0, per_tile, dump, 0)

    return pl.kernel(
        body,
        out_type=jax.ShapeDtypeStruct((_NC * N, H), jnp.float32),
        mesh=_mesh(),
        scratch_types=[
            pltpu.VMEM((EP,), jnp.int32),
            pltpu.VMEM((EP,), jnp.int32),
            pltpu.VMEM((EP,), jnp.float32),
            pltpu.VMEM((_B, H), jnp.float32),
            pltpu.VMEM((_B,), jnp.int32),
            pltpu.VMEM((_B,), jnp.int32),
            pltpu.VMEM((ZB, H), jnp.float32),
            pltpu.VMEM((DB, H), jnp.float32),
            pltpu.VMEM_SHARED((N, H), jnp.float32),
        ],
    )


def _inv_body(cnt_ref, inv_ref):
    c = cnt_ref[0] + cnt_ref[1]
    inv_ref[...] = 1.0 / jnp.maximum(c, 1.0)


def _make_act(N, R, H, BN):
    def body(p0, p1, r1, b1, w2, x_ref, t_ref):
        v = p0[...] + p1[...] + r1[...] + b1[...]
        x = jnp.where(v > 0, v, 0.01 * v)
        x_ref[...] = x
        for r in range(R):
            t_ref[r] = jnp.dot(x, w2[r], preferred_element_type=jnp.float32)

    return pl.pallas_call(
        body,
        grid=(N // BN,),
        in_specs=[
            pl.BlockSpec((BN, H), lambda i: (i, 0)),
            pl.BlockSpec((BN, H), lambda i: (i, 0)),
            pl.BlockSpec((BN, H), lambda i: (i, 0)),
            pl.BlockSpec((1, H), lambda i: (0, 0)),
            pl.BlockSpec((R, H, H), lambda i: (0, 0, 0)),
        ],
        out_specs=[
            pl.BlockSpec((BN, H), lambda i: (i, 0)),
            pl.BlockSpec((R, BN, H), lambda i: (0, i, 0)),
        ],
        out_shape=[
            jax.ShapeDtypeStruct((N, H), jnp.float32),
            jax.ShapeDtypeStruct((R, N, H), jnp.float32),
        ],
    )


def _make_out(N, H, BN):
    def body(p0, p1, x, r2, b2, o_ref):
        o_ref[...] = (p0[...] + p1[...] + b2[...]
                      + jnp.dot(x[...], r2[...],
                                preferred_element_type=jnp.float32))

    return pl.pallas_call(
        body,
        grid=(N // BN,),
        in_specs=[
            pl.BlockSpec((BN, H), lambda i: (i, 0)),
            pl.BlockSpec((BN, H), lambda i: (i, 0)),
            pl.BlockSpec((BN, H), lambda i: (i, 0)),
            pl.BlockSpec((H, H), lambda i: (0, 0)),
            pl.BlockSpec((1, H), lambda i: (0, 0)),
        ],
        out_specs=pl.BlockSpec((BN, H), lambda i: (i, 0)),
        out_shape=jax.ShapeDtypeStruct((N, H), jnp.float32),
    )


def kernel(edge_index, edge_type, weight1, root1, bias1, weight2, root2,
           bias2):
    R, N, H = weight1.shape
    E = edge_type.shape[0]
    NR = N * R
    assert E % (_NW * _B) == 0 and N % (_NS * 25) == 0 and NR % _NS == 0

    src = edge_index[0].astype(jnp.int32)
    dst = edge_index[1].astype(jnp.int32)
    typ = edge_type.astype(jnp.int32)

    k_hist = _make_hist(N, R, E)
    gidx, seg, cnt = k_hist(src, dst, typ)

    k_inv = pl.pallas_call(
        _inv_body,
        out_shape=jax.ShapeDtypeStruct((NR // 128, 128), jnp.float32))
    inv = k_inv(cnt.reshape(_NC, NR // 128, 128)).reshape(NR)

    k_w = _make_w(N, R, E)
    w = k_w(seg, inv)

    k_layer = _make_layer(N, R, H, E)
    part1 = k_layer(weight1.reshape(NR, H), gidx, dst, w)

    k_act = _make_act(N, R, H, 400)
    x, t = k_act(part1[:N], part1[N:], root1, bias1.reshape(1, H), weight2)

    part2 = k_layer(t.reshape(NR, H), gidx, dst, w)

    k_out = _make_out(N, H, 1000)
    return k_out(part2[:N], part2[N:], x, root2, bias2.reshape(1, H))
